# baseline (speedup 1.0000x reference)
import functools

import jax
import jax.numpy as jnp
from jax import lax
from jax.experimental import pallas as pl
from jax.experimental.pallas import tpu as pltpu
from jax.experimental.pallas import tpu_sc as plsc

_K = 16
_RADIUS = 0.8
_BINS = 50
_WIDTH = 0.2
_L = 16
_NC = 2
_NS = 16
_GROUP = 8
_SC_FRACTION = 0.46875


_GATHER_DNUMS = lax.GatherDimensionNumbers(
    offset_dims=(), collapsed_slice_dims=(0,), start_index_map=(0,))


def _tc_select_kernel(d_ref, cols_ref, vals_ref):
    d = d_ref[...]
    r, n = d.shape
    colidx = lax.broadcasted_iota(jnp.int32, (r, n), 1)
    inf_bits = jnp.int32(0x7F800000)
    dead = jnp.int32(0x7FFFFFFF)
    kint = jnp.where(d <= _RADIUS, lax.bitcast_convert_type(d, jnp.int32),
                     inf_bits)
    cols_list = []
    vbits_list = []
    for _ in range(_K):
        v = jnp.min(kint, axis=1, keepdims=True)
        cand = jnp.where(kint == v, colidx, n)
        idx = jnp.min(cand, axis=1, keepdims=True)
        kint = jnp.where(colidx == idx, dead, kint)
        cols_list.append(idx)
        vbits_list.append(v)
    cols = jnp.concatenate(cols_list, axis=1)
    vbits = jnp.concatenate(vbits_list, axis=1)

    def fix_degenerate():
        outs = []
        for k in range(_K):
            hit = colidx == cols[:, k:k + 1]
            outs.append(jnp.sum(jnp.where(hit, d, 0.0), axis=1, keepdims=True))
        return jnp.concatenate(outs, axis=1)

    vals = lax.cond(jnp.any(vbits == inf_bits), fix_degenerate,
                    lambda: lax.bitcast_convert_type(vbits, jnp.float32))
    cols_ref[...] = cols
    vals_ref[...] = vals


def _vgather(x, idx):
    return lax.gather(x, idx[:, None], _GATHER_DNUMS, (1,),
                      mode=lax.GatherScatterMode.PROMISE_IN_BOUNDS)


def _sc_select(n, nsc, distance_matrix):
    nw = _NC * _NS
    rows_per = nsc // nw
    nchunk = n // _L
    mesh = plsc.VectorSubcoreMesh(core_axis_name="c", subcore_axis_name="s")

    @functools.partial(
        pl.kernel,
        mesh=mesh,
        out_type=[
            jax.ShapeDtypeStruct((nsc * _K,), jnp.int32),
            jax.ShapeDtypeStruct((nsc * _K,), jnp.float32),
        ],
        scratch_types=[
            pltpu.VMEM((2, n), jnp.float32),
            pltpu.VMEM((rows_per * _K,), jnp.int32),
            pltpu.VMEM((rows_per * _K,), jnp.float32),
            pltpu.SemaphoreType.DMA,
            pltpu.SemaphoreType.DMA,
        ],
        compiler_params=pltpu.CompilerParams(needs_layout_passes=False),
    )
    def sel(d_hbm, cols_hbm, vals_hbm, rowbuf, colbuf, valbuf, sem0, sem1):
        wid = lax.axis_index("s") * _NC + lax.axis_index("c")
        row0 = wid * rows_per
        iota = lax.iota(jnp.int32, _L)
        fifteen = jnp.full((_L,), _L - 1, jnp.int32)
        inf_v = jnp.full((_L,), jnp.inf, jnp.float32)
        big_i = jnp.full((_L,), jnp.int32(2**31 - 1), jnp.int32)
        shift_idx = jnp.maximum(iota - 1, 0)

        def scan_row(b, t0):
            def group_body(g, st):
                bk, bi, trun = st
                base = g * (_GROUP * _L)
                vs = [rowbuf[b, pl.ds(base + j * _L, _L)] for j in range(_GROUP)]
                anyc = vs[0] <= trun
                for j in range(1, _GROUP):
                    anyc = jnp.logical_or(anyc, vs[j] <= trun)

                def chunk_insert(st, j):
                    v = vs[j]
                    colv = iota + (base + j * _L)
                    m0 = v <= st[2]

                    def ins_cond(s):
                        return plsc.all_reduce_population_count(s[3])[0] > 0

                    def ins_body(s, v=v, colv=colv):
                        bk, bi, trun, m = s
                        lane = plsc.all_reduce_ffs(m)
                        vc = _vgather(v, lane)
                        keyc = jnp.where(vc <= _RADIUS, vc, jnp.inf)
                        colc = _vgather(colv, lane)
                        lt = jnp.logical_or(
                            bk < keyc,
                            jnp.logical_and(bk == keyc, bi < colc))
                        pos = plsc.all_reduce_population_count(lt)
                        shk = _vgather(bk, shift_idx)
                        shi = _vgather(bi, shift_idx)
                        keep = iota < pos
                        ins = iota == pos
                        bk = jnp.where(keep, bk, jnp.where(ins, keyc, shk))
                        bi = jnp.where(keep, bi, jnp.where(ins, colc, shi))
                        trun = jnp.minimum(t0, _vgather(bk, fifteen))
                        m = jnp.logical_and(m, iota != lane)
                        m = jnp.logical_and(m, v <= trun)
                        return (bk, bi, trun, m)

                    out = lax.while_loop(
                        ins_cond, ins_body, (st[0], st[1], st[2], m0))
                    return (out[0], out[1], out[2])

                def do_insert(st):
                    for h in range(2):
                        js = range(h * (_GROUP // 2), (h + 1) * (_GROUP // 2))
                        hm = vs[js[0]] <= st[2]
                        for j in js[1:]:
                            hm = jnp.logical_or(hm, vs[j] <= st[2])

                        def half_fn(st, js=js):
                            for j in js:
                                st = chunk_insert(st, j)
                            return st

                        st = lax.cond(
                            plsc.all_reduce_population_count(hm)[0] > 0,
                            half_fn, lambda s: s, st)
                    return st

                n_cand = plsc.all_reduce_population_count(anyc)[0]
                return lax.cond(n_cand > 0, do_insert, lambda s: s,
                                (bk, bi, trun))

            bk, bi, _t = lax.fori_loop(
                0, nchunk // _GROUP, group_body, (inf_v, big_i, t0))
            return bk, bi

        def process_row(b, ri):
            def min_body(c, accs):
                base = c * (16 * _L)
                half = [
                    jnp.minimum(accs[j], rowbuf[b, pl.ds(base + j * _L, _L)])
                    for j in range(8)]
                return tuple(
                    jnp.minimum(half[j],
                                rowbuf[b, pl.ds(base + (8 + j) * _L, _L)])
                    for j in range(8))

            accs = lax.fori_loop(0, n // (16 * _L), min_body, (inf_v,) * 8)
            rev = (_L - 1) - iota

            def merge_lo16(x, y):
                return jnp.minimum(x, _vgather(y, rev))

            def vsorted(x):
                return plsc.sort_key_val(x, x)[0]

            ms = [jnp.minimum(accs[2 * j], accs[2 * j + 1]) for j in range(4)]
            lo_ab = merge_lo16(vsorted(ms[0]), vsorted(ms[1]))
            lo_cd = merge_lo16(vsorted(ms[2]), vsorted(ms[3]))
            lo = merge_lo16(vsorted(lo_ab), vsorted(lo_cd))
            for d in (8, 4, 2, 1):
                lo = jnp.maximum(lo, _vgather(lo, iota ^ d))
            bk, bi = scan_row(b, lo)
            kd = _vgather(bk, fifteen)
            bk, bi = lax.cond(
                kd[0] > _RADIUS, lambda: scan_row(b, inf_v), lambda: (bk, bi))
            vals = plsc.load_gather(
                rowbuf, [jnp.full((_L,), b, jnp.int32), bi])
            colbuf[pl.ds(ri * _K, _K)] = bi
            valbuf[pl.ds(ri * _K, _K)] = vals

        row_max = row0 + rows_per - 1
        pltpu.async_copy(d_hbm.at[row0], rowbuf.at[0], sem0)

        def pair_body(k, carry):
            r0 = row0 + 2 * k
            pltpu.async_copy(d_hbm.at[r0 + 1], rowbuf.at[1], sem1)
            pltpu.make_async_copy(d_hbm.at[r0], rowbuf.at[0], sem0).wait()
            process_row(0, 2 * k)
            pltpu.async_copy(
                d_hbm.at[jnp.minimum(r0 + 2, row_max)], rowbuf.at[0], sem0)
            pltpu.make_async_copy(d_hbm.at[r0 + 1], rowbuf.at[1], sem1).wait()
            process_row(1, 2 * k + 1)
            return carry

        lax.fori_loop(0, rows_per // 2, pair_body, 0)
        pltpu.make_async_copy(d_hbm.at[row_max], rowbuf.at[0], sem0).wait()
        pltpu.sync_copy(colbuf, cols_hbm.at[pl.ds(row0 * _K, rows_per * _K)])
        pltpu.sync_copy(valbuf, vals_hbm.at[pl.ds(row0 * _K, rows_per * _K)])

    return sel(distance_matrix)


def _feature_kernel(v_ref, feat_ref):
    dflat = v_ref[...]
    centers = lax.broadcasted_iota(jnp.int32, (1, _BINS), 1).astype(jnp.float32) * (
        1.0 / (_BINS - 1))
    z = (dflat - centers) * (1.0 / _WIDTH)
    feat_ref[...] = jnp.exp(-0.5 * z * z)


def kernel(distance_matrix):
    n = distance_matrix.shape[0]
    nsc = (int(n * _SC_FRACTION) // (_NC * _NS * 2)) * (_NC * _NS * 2)
    ntc = n - nsc
    sc_cols, sc_vals = _sc_select(n, nsc, distance_matrix)
    r = min(128, ntc)
    tc_cols, tc_vals = pl.pallas_call(
        _tc_select_kernel,
        grid=(ntc // r,),
        in_specs=[pl.BlockSpec((r, n), lambda i, o=nsc // r: (i + o, 0))],
        out_specs=[
            pl.BlockSpec((r, _K), lambda i: (i, 0)),
            pl.BlockSpec((r, _K), lambda i: (i, 0)),
        ],
        out_shape=[
            jax.ShapeDtypeStruct((ntc, _K), jnp.int32),
            jax.ShapeDtypeStruct((ntc, _K), jnp.float32),
        ],
    )(distance_matrix)
    cols = jnp.concatenate([sc_cols.reshape(nsc, _K), tc_cols], axis=0)

    def expand(v):
        e = v.shape[0] * _K
        fb = min(8192, e)
        return pl.pallas_call(
            _feature_kernel,
            grid=(e // fb,),
            in_specs=[pl.BlockSpec((fb, 1), lambda i: (i, 0))],
            out_specs=pl.BlockSpec((fb, _BINS), lambda i: (i, 0)),
            out_shape=jax.ShapeDtypeStruct((e, _BINS), jnp.float32),
        )(v.reshape(e, 1))

    feats = jnp.concatenate(
        [expand(sc_vals.reshape(nsc, _K)), expand(tc_vals)], axis=0)
    rows = jnp.broadcast_to(jnp.arange(n, dtype=cols.dtype)[:, None], (n, _K))
    edge_index = jnp.stack([rows.reshape(-1), cols.reshape(-1)], axis=1)
    return edge_index, feats


# speedup vs baseline: 1.1140x; 1.1140x over previous
import functools

import jax
import jax.numpy as jnp
from jax import lax
from jax.experimental import pallas as pl
from jax.experimental.pallas import tpu as pltpu
from jax.experimental.pallas import tpu_sc as plsc

_K = 16
_RADIUS = 0.8
_BINS = 50
_WIDTH = 0.2
_L = 16
_NC = 2
_NS = 16
_GROUP = 8
_SC_FRACTION = 0.5


_GATHER_DNUMS = lax.GatherDimensionNumbers(
    offset_dims=(), collapsed_slice_dims=(0,), start_index_map=(0,))


def _tc_select_kernel(d_ref, cols_ref, vals_ref):
    d = d_ref[...]
    r, n = d.shape
    colidx = lax.broadcasted_iota(jnp.int32, (r, n), 1)
    inf_bits = jnp.int32(0x7F800000)
    dead = jnp.int32(0x7FFFFFFF)
    kint = jnp.where(d <= _RADIUS, lax.bitcast_convert_type(d, jnp.int32),
                     inf_bits)
    cols_list = []
    vbits_list = []
    for _ in range(_K):
        v = jnp.min(kint, axis=1, keepdims=True)
        cand = jnp.where(kint == v, colidx, n)
        idx = jnp.min(cand, axis=1, keepdims=True)
        kint = jnp.where(colidx == idx, dead, kint)
        cols_list.append(idx)
        vbits_list.append(v)
    cols = jnp.concatenate(cols_list, axis=1)
    vbits = jnp.concatenate(vbits_list, axis=1)

    def fix_degenerate():
        outs = []
        for k in range(_K):
            hit = colidx == cols[:, k:k + 1]
            outs.append(jnp.sum(jnp.where(hit, d, 0.0), axis=1, keepdims=True))
        return jnp.concatenate(outs, axis=1)

    vals = lax.cond(jnp.any(vbits == inf_bits), fix_degenerate,
                    lambda: lax.bitcast_convert_type(vbits, jnp.float32))
    cols_ref[...] = cols
    vals_ref[...] = vals


def _vgather(x, idx):
    return lax.gather(x, idx[:, None], _GATHER_DNUMS, (1,),
                      mode=lax.GatherScatterMode.PROMISE_IN_BOUNDS)


def _sc_select(n, nsc, distance_matrix):
    nw = _NC * _NS
    rows_per = nsc // nw
    nchunk = n // _L
    mesh = plsc.VectorSubcoreMesh(core_axis_name="c", subcore_axis_name="s")

    @functools.partial(
        pl.kernel,
        mesh=mesh,
        out_type=[
            jax.ShapeDtypeStruct((nsc * _K,), jnp.int32),
            jax.ShapeDtypeStruct((nsc * _K,), jnp.float32),
        ],
        scratch_types=[
            pltpu.VMEM((2, n), jnp.float32),
            pltpu.VMEM((rows_per * _K,), jnp.int32),
            pltpu.VMEM((rows_per * _K,), jnp.float32),
            pltpu.SemaphoreType.DMA,
            pltpu.SemaphoreType.DMA,
        ],
        compiler_params=pltpu.CompilerParams(needs_layout_passes=False),
    )
    def sel(d_hbm, cols_hbm, vals_hbm, rowbuf, colbuf, valbuf, sem0, sem1):
        wid = lax.axis_index("s") * _NC + lax.axis_index("c")
        row0 = wid * rows_per
        iota = lax.iota(jnp.int32, _L)
        fifteen = jnp.full((_L,), _L - 1, jnp.int32)
        inf_v = jnp.full((_L,), jnp.inf, jnp.float32)
        big_i = jnp.full((_L,), jnp.int32(2**31 - 1), jnp.int32)
        shift_idx = jnp.maximum(iota - 1, 0)

        def scan_row(b, t0):
            def group_body(g, st):
                bk, bi, trun = st
                base = g * (_GROUP * _L)
                vs = [rowbuf[b, pl.ds(base + j * _L, _L)] for j in range(_GROUP)]
                anyc = vs[0] <= trun
                for j in range(1, _GROUP):
                    anyc = jnp.logical_or(anyc, vs[j] <= trun)

                def chunk_insert(st, j):
                    v = vs[j]
                    colv = iota + (base + j * _L)
                    m0 = v <= st[2]

                    def ins_cond(s):
                        return plsc.all_reduce_population_count(s[3])[0] > 0

                    def ins_body(s, v=v, colv=colv):
                        bk, bi, trun, m = s
                        lane = plsc.all_reduce_ffs(m)
                        vc = _vgather(v, lane)
                        keyc = jnp.where(vc <= _RADIUS, vc, jnp.inf)
                        colc = _vgather(colv, lane)
                        lt = jnp.logical_or(
                            bk < keyc,
                            jnp.logical_and(bk == keyc, bi < colc))
                        pos = plsc.all_reduce_population_count(lt)
                        shk = _vgather(bk, shift_idx)
                        shi = _vgather(bi, shift_idx)
                        keep = iota < pos
                        ins = iota == pos
                        bk = jnp.where(keep, bk, jnp.where(ins, keyc, shk))
                        bi = jnp.where(keep, bi, jnp.where(ins, colc, shi))
                        trun = jnp.minimum(t0, _vgather(bk, fifteen))
                        m = jnp.logical_and(m, iota != lane)
                        m = jnp.logical_and(m, v <= trun)
                        return (bk, bi, trun, m)

                    out = lax.while_loop(
                        ins_cond, ins_body, (st[0], st[1], st[2], m0))
                    return (out[0], out[1], out[2])

                def do_insert(st):
                    for h in range(2):
                        js = range(h * (_GROUP // 2), (h + 1) * (_GROUP // 2))
                        hm = vs[js[0]] <= st[2]
                        for j in js[1:]:
                            hm = jnp.logical_or(hm, vs[j] <= st[2])

                        def half_fn(st, js=js):
                            for j in js:
                                st = chunk_insert(st, j)
                            return st

                        st = lax.cond(
                            plsc.all_reduce_population_count(hm)[0] > 0,
                            half_fn, lambda s: s, st)
                    return st

                n_cand = plsc.all_reduce_population_count(anyc)[0]
                return lax.cond(n_cand > 0, do_insert, lambda s: s,
                                (bk, bi, trun))

            bk, bi, _t = lax.fori_loop(
                0, nchunk // _GROUP, group_body, (inf_v, big_i, t0))
            return bk, bi

        def process_row(b, ri):
            def min_body(c, accs):
                base = c * (16 * _L)
                half = [
                    jnp.minimum(accs[j], rowbuf[b, pl.ds(base + j * _L, _L)])
                    for j in range(8)]
                return tuple(
                    jnp.minimum(half[j],
                                rowbuf[b, pl.ds(base + (8 + j) * _L, _L)])
                    for j in range(8))

            accs = lax.fori_loop(0, n // (16 * _L), min_body, (inf_v,) * 8)
            rev = (_L - 1) - iota

            def merge_lo16(x, y):
                return jnp.minimum(x, _vgather(y, rev))

            def vsorted(x):
                return plsc.sort_key_val(x, x)[0]

            ms = [jnp.minimum(accs[2 * j], accs[2 * j + 1]) for j in range(4)]
            lo_ab = merge_lo16(vsorted(ms[0]), vsorted(ms[1]))
            lo_cd = merge_lo16(vsorted(ms[2]), vsorted(ms[3]))
            lo = merge_lo16(vsorted(lo_ab), vsorted(lo_cd))
            for d in (8, 4, 2, 1):
                lo = jnp.maximum(lo, _vgather(lo, iota ^ d))
            bk, bi = scan_row(b, lo)
            kd = _vgather(bk, fifteen)
            bk, bi = lax.cond(
                kd[0] > _RADIUS, lambda: scan_row(b, inf_v), lambda: (bk, bi))
            vals = plsc.load_gather(
                rowbuf, [jnp.full((_L,), b, jnp.int32), bi])
            colbuf[pl.ds(ri * _K, _K)] = bi
            valbuf[pl.ds(ri * _K, _K)] = vals

        row_max = row0 + rows_per - 1
        pltpu.async_copy(d_hbm.at[row0], rowbuf.at[0], sem0)

        def pair_body(k, carry):
            r0 = row0 + 2 * k
            pltpu.async_copy(d_hbm.at[r0 + 1], rowbuf.at[1], sem1)
            pltpu.make_async_copy(d_hbm.at[r0], rowbuf.at[0], sem0).wait()
            process_row(0, 2 * k)
            pltpu.async_copy(
                d_hbm.at[jnp.minimum(r0 + 2, row_max)], rowbuf.at[0], sem0)
            pltpu.make_async_copy(d_hbm.at[r0 + 1], rowbuf.at[1], sem1).wait()
            process_row(1, 2 * k + 1)
            return carry

        lax.fori_loop(0, rows_per // 2, pair_body, 0)
        pltpu.make_async_copy(d_hbm.at[row_max], rowbuf.at[0], sem0).wait()
        pltpu.sync_copy(colbuf, cols_hbm.at[pl.ds(row0 * _K, rows_per * _K)])
        pltpu.sync_copy(valbuf, vals_hbm.at[pl.ds(row0 * _K, rows_per * _K)])

    return sel(distance_matrix)


def _feature_kernel(v_ref, feat_ref):
    dflat = v_ref[...]
    centers = lax.broadcasted_iota(jnp.int32, (1, _BINS), 1).astype(jnp.float32) * (
        1.0 / (_BINS - 1))
    z = (dflat - centers) * (1.0 / _WIDTH)
    feat_ref[...] = jnp.exp(-0.5 * z * z)


def kernel(distance_matrix):
    n = distance_matrix.shape[0]
    nsc = (int(n * _SC_FRACTION) // (_NC * _NS * 2)) * (_NC * _NS * 2)
    ntc = n - nsc
    sc_cols, sc_vals = _sc_select(n, nsc, distance_matrix)
    r = min(256, ntc)
    tc_cols, tc_vals = pl.pallas_call(
        _tc_select_kernel,
        grid=(ntc // r,),
        in_specs=[pl.BlockSpec((r, n), lambda i, o=nsc // r: (i + o, 0))],
        out_specs=[
            pl.BlockSpec((r, _K), lambda i: (i, 0)),
            pl.BlockSpec((r, _K), lambda i: (i, 0)),
        ],
        out_shape=[
            jax.ShapeDtypeStruct((ntc, _K), jnp.int32),
            jax.ShapeDtypeStruct((ntc, _K), jnp.float32),
        ],
    )(distance_matrix)
    cols = jnp.concatenate([sc_cols.reshape(nsc, _K), tc_cols], axis=0)

    def expand(v):
        e = v.shape[0] * _K
        fb = min(8192, e)
        return pl.pallas_call(
            _feature_kernel,
            grid=(e // fb,),
            in_specs=[pl.BlockSpec((fb, 1), lambda i: (i, 0))],
            out_specs=pl.BlockSpec((fb, _BINS), lambda i: (i, 0)),
            out_shape=jax.ShapeDtypeStruct((e, _BINS), jnp.float32),
        )(v.reshape(e, 1))

    feats = jnp.concatenate(
        [expand(sc_vals.reshape(nsc, _K)), expand(tc_vals)], axis=0)
    rows = jnp.broadcast_to(jnp.arange(n, dtype=cols.dtype)[:, None], (n, _K))
    edge_index = jnp.stack([rows.reshape(-1), cols.reshape(-1)], axis=1)
    return edge_index, feats
